# baseline (speedup 1.0000x reference)
import functools

import jax
import jax.numpy as jnp
from jax import lax
from jax.experimental import pallas as pl
from jax.experimental.pallas import tpu as pltpu
from jax.experimental.pallas import tpu_sc as plsc

BATCH = 16384
DIM_HEAD = 64
DIM_MID = 32
DIM_TAIL = 16

_INFO = plsc.get_sparse_core_info()
NC, NS = _INFO.num_cores, _INFO.num_subcores
NW = NC * NS
N_PER = BATCH // NW
CH = 128
NT = N_PER // CH
NSTEP = N_PER // 16
OUT_ROWS = BATCH + 8
DUMMY_ROW = BATCH


def _body(x_hbm, g_hbm, head_hbm, mid_hbm, tail_hbm, out_hbm,
          xv, gv, xk0, pk0, xk1, pk1, xk2, pk2,
          gb_mid, gb_tail, ob_head, ob_mid, ob_tail, sem_g, sem_s):
    wid = lax.axis_index("s") * NC + lax.axis_index("c")
    base = wid * N_PER
    pltpu.sync_copy(x_hbm.at[pl.ds(base, N_PER)], xv)
    pltpu.sync_copy(g_hbm.at[pl.ds(base, N_PER)], gv)

    zf = jnp.zeros((16,), jnp.float32)
    zi = jnp.zeros((16,), jnp.int32)
    dummy = jnp.full((16,), DUMMY_ROW, jnp.int32)

    for r in range(N_PER):
        ob_mid[r, pl.ds(DIM_MID, 16)] = zf
        ob_mid[r, pl.ds(DIM_MID + 16, 16)] = zf

    for t in range(NT):
        for c2 in range(CH // 16):
            for xk, pk in ((xk0, pk0), (xk1, pk1), (xk2, pk2)):
                xk[t, pl.ds(c2 * 16, 16)] = zi
                pk[t, pl.ds(c2 * 16, 16)] = dummy

    iota = lax.iota(jnp.int32, 16)
    offs = [jnp.int32(0), jnp.int32(0), jnp.int32(0)]
    for c in range(NSTEP):
        xc = xv[pl.ds(c * 16, 16)]
        gc = gv[pl.ds(c * 16, 16)]
        posc = iota + (base + c * 16)
        for k, (xk, pk) in enumerate(((xk0, pk0), (xk1, pk1), (xk2, pk2))):
            m = gc == k
            ones = m.astype(jnp.int32)
            incl = plsc.cumsum(ones)
            dest = offs[k] + incl - ones
            rows = lax.shift_right_logical(dest, 7)
            cols = lax.bitwise_and(dest, CH - 1)
            plsc.store_scatter(xk, [rows, cols], xc, mask=m)
            plsc.store_scatter(pk, [rows, cols], posc, mask=m)
            offs[k] = offs[k] + jnp.sum(ones)

    def run_group(nk, xk, pk, tbl, gbuf, obuf, widen, dg):
        trips = lax.shift_right_logical(nk + (CH - 1), 7)

        def fire_g(j, carry):
            pltpu.async_copy(tbl.at[xk.at[j]], gbuf.at[pl.ds(j * CH, CH)],
                             sem_g)
            return carry

        def drain_g(j, carry):
            pltpu.make_async_copy(tbl.at[xk.at[j]],
                                  gbuf.at[pl.ds(j * CH, CH)], sem_g).wait()
            return carry

        def widen_j(j, carry):
            widen(j * CH)
            return carry

        def fire_s(j, carry):
            pltpu.async_copy(obuf.at[pl.ds(j * CH, CH)],
                             out_hbm.at[pk.at[j]], sem_s)
            return carry

        def drain_s(j, carry):
            pltpu.make_async_copy(obuf.at[pl.ds(j * CH, CH)],
                                  out_hbm.at[pk.at[j]], sem_s).wait()
            return carry

        lax.fori_loop(0, trips, fire_g, jnp.int32(0))
        lax.fori_loop(0, trips, drain_g, jnp.int32(0))
        if widen is not None:
            lax.fori_loop(0, trips, widen_j, jnp.int32(0))
        lax.fori_loop(0, trips, fire_s, jnp.int32(0))
        lax.fori_loop(0, trips, drain_s, jnp.int32(0))

    def widen_mid(jb):
        for r in range(CH):
            ob_mid[jb + r, pl.ds(0, 16)] = gb_mid[jb + r, pl.ds(0, 16)]
            ob_mid[jb + r, pl.ds(16, 16)] = gb_mid[jb + r, pl.ds(16, 16)]

    def widen_tail(jb):
        for r in range(CH):
            t = gb_tail[jb + r, pl.ds(0, 16)]
            for q in range(4):
                ob_tail[jb + r, pl.ds(q * 16, 16)] = t

    run_group(offs[0], xk0, pk0, head_hbm, ob_head, ob_head, None, 0)
    run_group(offs[1], xk1, pk1, mid_hbm, gb_mid, ob_mid, widen_mid, 1)
    run_group(offs[2], xk2, pk2, tail_hbm, gb_tail, ob_tail, widen_tail, 2)


@jax.jit
def _sc_lookup(x, g, head_table, mid_table, tail_table):
    mesh = plsc.VectorSubcoreMesh(core_axis_name="c", subcore_axis_name="s")
    f = functools.partial(
        pl.kernel,
        mesh=mesh,
        compiler_params=pltpu.CompilerParams(
            needs_layout_passes=False, use_tc_tiling_on_sc=False),
        out_type=jax.ShapeDtypeStruct((OUT_ROWS, DIM_HEAD), jnp.float32),
        scratch_types=[
            pltpu.VMEM((N_PER,), jnp.int32),
            pltpu.VMEM((N_PER,), jnp.int32),
            pltpu.VMEM((NT, CH), jnp.int32),
            pltpu.VMEM((NT, CH), jnp.int32),
            pltpu.VMEM((NT, CH), jnp.int32),
            pltpu.VMEM((NT, CH), jnp.int32),
            pltpu.VMEM((NT, CH), jnp.int32),
            pltpu.VMEM((NT, CH), jnp.int32),
            pltpu.VMEM((N_PER, DIM_MID), jnp.float32),
            pltpu.VMEM((N_PER, DIM_TAIL), jnp.float32),
            pltpu.VMEM((N_PER, DIM_HEAD), jnp.float32),
            pltpu.VMEM((N_PER, DIM_HEAD), jnp.float32),
            pltpu.VMEM((N_PER, DIM_HEAD), jnp.float32),
            pltpu.SemaphoreType.DMA,
            pltpu.SemaphoreType.DMA,
        ],
    )(_body)
    return f(x, g, head_table, mid_table, tail_table)


def kernel(x, frequency_groups, head_table, mid_table, tail_table):
    out = _sc_lookup(x.astype(jnp.int32), frequency_groups.astype(jnp.int32),
                     head_table, mid_table[:100000], tail_table)
    return out[:BATCH]


# speedup vs baseline: 1.7939x; 1.7939x over previous
import functools

import jax
import jax.numpy as jnp
from jax import lax
from jax.experimental import pallas as pl
from jax.experimental.pallas import tpu as pltpu
from jax.experimental.pallas import tpu_sc as plsc

BATCH = 16384
DIM_HEAD = 64
DIM_MID = 32
DIM_TAIL = 16

_INFO = plsc.get_sparse_core_info()
NC, NS = _INFO.num_cores, _INFO.num_subcores
NW = NC * NS
N_PER = BATCH // NW
CH = 32
NSTEP = N_PER // 16


def _body(x_hbm, g_hbm, head_hbm, mid_hbm, tail_hbm, out_hbm,
          xv, gv, xk0, pk0, xk1, pk1, xk2, pk2,
          gb_head, gb_mid, gb_tail, obuf, sem0, sem1, sem2):
    wid = lax.axis_index("s") * NC + lax.axis_index("c")
    base = wid * N_PER
    pltpu.sync_copy(x_hbm.at[pl.ds(base, N_PER)], xv)
    pltpu.sync_copy(g_hbm.at[pl.ds(base, N_PER)], gv)

    zi = jnp.zeros((16,), jnp.int32)
    zf = jnp.zeros((16,), jnp.float32)

    for i in range(NSTEP):
        xk0[pl.ds(i * 16, 16)] = zi
        xk1[pl.ds(i * 16, 16)] = zi
        xk2[pl.ds(i * 16, 16)] = zi

    iota = lax.iota(jnp.int32, 16)
    offs = [jnp.int32(0), jnp.int32(0), jnp.int32(0)]
    for c in range(NSTEP):
        xc = xv[pl.ds(c * 16, 16)]
        gc = gv[pl.ds(c * 16, 16)]
        posc = iota + (c * 16)
        for k, (xk, pk) in enumerate(((xk0, pk0), (xk1, pk1), (xk2, pk2))):
            m = gc == k
            ones = m.astype(jnp.int32)
            incl = plsc.cumsum(ones)
            dest = offs[k] + incl - ones
            plsc.store_scatter(xk, [dest], xc, mask=m)
            plsc.store_scatter(pk, [dest], posc, mask=m)
            offs[k] = offs[k] + jnp.sum(ones)

    def fire_all(nk, xk, tbl, gbuf, sem):
        trips = lax.shift_right_logical(nk + (CH - 1), 5)

        def fire(j, carry):
            pltpu.async_copy(tbl.at[xk.at[pl.ds(j * CH, CH)]],
                             gbuf.at[pl.ds(j * CH, CH)], sem)
            return carry

        lax.fori_loop(0, trips, fire, jnp.int32(0))
        return trips

    def drain_all(trips, xk, tbl, gbuf, sem):
        def drain(j, carry):
            pltpu.make_async_copy(tbl.at[xk.at[pl.ds(j * CH, CH)]],
                                  gbuf.at[pl.ds(j * CH, CH)], sem).wait()
            return carry

        lax.fori_loop(0, trips, drain, jnp.int32(0))

    t0 = fire_all(offs[0], xk0, head_hbm, gb_head, sem0)
    t1 = fire_all(offs[1], xk1, mid_hbm, gb_mid, sem1)
    t2 = fire_all(offs[2], xk2, tail_hbm, gb_tail, sem2)

    def reloc(nk, pk, emit16):
        nsteps = lax.shift_right_logical(nk + 15, 4)

        def step(j, carry):
            rows = iota + j * 16
            mask = rows < nk
            pos = plsc.load_gather(pk, [rows])
            pos = lax.bitwise_and(pos, N_PER - 1)
            emit16(rows, pos, mask)
            return carry

        lax.fori_loop(0, nsteps, step, jnp.int32(0))

    def head16(rows, pos, mask):
        for c in range(DIM_HEAD):
            cv = jnp.full((16,), c, jnp.int32)
            v = plsc.load_gather(gb_head, [rows, cv])
            plsc.store_scatter(obuf, [pos, cv], v, mask=mask)

    def mid16(rows, pos, mask):
        for c in range(DIM_MID):
            cv = jnp.full((16,), c, jnp.int32)
            v = plsc.load_gather(gb_mid, [rows, cv])
            plsc.store_scatter(obuf, [pos, cv], v, mask=mask)
        for c in range(DIM_MID, DIM_HEAD):
            cv = jnp.full((16,), c, jnp.int32)
            plsc.store_scatter(obuf, [pos, cv], zf, mask=mask)

    def tail16(rows, pos, mask):
        for c in range(DIM_TAIL):
            cv = jnp.full((16,), c, jnp.int32)
            v = plsc.load_gather(gb_tail, [rows, cv])
            for q in range(DIM_HEAD // DIM_TAIL):
                cq = jnp.full((16,), c + q * DIM_TAIL, jnp.int32)
                plsc.store_scatter(obuf, [pos, cq], v, mask=mask)

    drain_all(t0, xk0, head_hbm, gb_head, sem0)
    reloc(offs[0], pk0, head16)
    drain_all(t1, xk1, mid_hbm, gb_mid, sem1)
    reloc(offs[1], pk1, mid16)
    drain_all(t2, xk2, tail_hbm, gb_tail, sem2)
    reloc(offs[2], pk2, tail16)

    pltpu.sync_copy(obuf, out_hbm.at[pl.ds(base, N_PER)])


@jax.jit
def _sc_lookup(x, g, head_table, mid_table, tail_table):
    mesh = plsc.VectorSubcoreMesh(core_axis_name="c", subcore_axis_name="s")
    f = functools.partial(
        pl.kernel,
        mesh=mesh,
        compiler_params=pltpu.CompilerParams(
            needs_layout_passes=False, use_tc_tiling_on_sc=False),
        out_type=jax.ShapeDtypeStruct((BATCH, DIM_HEAD), jnp.float32),
        scratch_types=[
            pltpu.VMEM((N_PER,), jnp.int32),
            pltpu.VMEM((N_PER,), jnp.int32),
            pltpu.VMEM((N_PER,), jnp.int32),
            pltpu.VMEM((N_PER,), jnp.int32),
            pltpu.VMEM((N_PER,), jnp.int32),
            pltpu.VMEM((N_PER,), jnp.int32),
            pltpu.VMEM((N_PER,), jnp.int32),
            pltpu.VMEM((N_PER,), jnp.int32),
            pltpu.VMEM((N_PER, DIM_HEAD), jnp.float32),
            pltpu.VMEM((N_PER, DIM_MID), jnp.float32),
            pltpu.VMEM((N_PER, DIM_TAIL), jnp.float32),
            pltpu.VMEM((N_PER, DIM_HEAD), jnp.float32),
            pltpu.SemaphoreType.DMA,
            pltpu.SemaphoreType.DMA,
            pltpu.SemaphoreType.DMA,
        ],
    )(_body)
    return f(x, g, head_table, mid_table, tail_table)


def kernel(x, frequency_groups, head_table, mid_table, tail_table):
    return _sc_lookup(x.astype(jnp.int32), frequency_groups.astype(jnp.int32),
                      head_table, mid_table[:100000], tail_table)
